# Initial kernel scaffold; baseline (speedup 1.0000x reference)
import jax
import jax.numpy as jnp
from jax.experimental import pallas as pl


def kernel(X, edge_index, W1_0, b1_0, W1_1, b1_1, W2_0, b2_0, W2_1, b2_1, P0_W, P0_b, P1_W, P1_b, P2_W, P2_b):
    raise NotImplementedError("write your pallas kernel here")


# speedup vs baseline: 6.3265x; 6.3265x over previous
import functools

import jax
import jax.numpy as jnp
from jax import lax
from jax.experimental import pallas as pl
from jax.experimental.pallas import tpu as pltpu
from jax.experimental.pallas import tpu_sc as plsc

N = 10000
D = 128
E = 320000
C = 32

NC = 2
NS = 16
NW = NC * NS
EPW = E // NW
K = 80
NCHUNK = EPW // K
RPT = 624
TAIL = N - NS * RPT

_mesh = plsc.VectorSubcoreMesh(core_axis_name="c", subcore_axis_name="s")


@functools.partial(
    pl.kernel,
    mesh=_mesh,
    out_type=jax.ShapeDtypeStruct((NC, N, D), jnp.float32),
    scratch_types=[
        pltpu.VMEM((NCHUNK, K), jnp.int32),
        pltpu.VMEM((NCHUNK, K), jnp.int32),
        pltpu.VMEM((K, D), jnp.float32),
        pltpu.VMEM((K, D), jnp.float32),
        pltpu.VMEM_SHARED((N, D), jnp.float32),
        pltpu.SemaphoreType.DMA,
        pltpu.SemaphoreType.DMA,
    ],
)
def _sc_scatter(src_hbm, dst_hbm, h_hbm, z_hbm, out_hbm,
                srcs_v, dsts_v, rows_a, rows_b, agg_sh, sem_a, sem_b):
    c = lax.axis_index("c")
    s = lax.axis_index("s")
    wid = c * NS + s
    pltpu.sync_copy(src_hbm.at[wid], srcs_v)
    pltpu.sync_copy(dst_hbm.at[wid], dsts_v)
    r0 = s * RPT
    pltpu.sync_copy(z_hbm.at[pl.ds(r0, RPT)], agg_sh.at[pl.ds(r0, RPT)])

    @pl.when(s == 0)
    def _zero_tail():
        pltpu.sync_copy(z_hbm.at[pl.ds(NS * RPT, TAIL)],
                        agg_sh.at[pl.ds(NS * RPT, TAIL)])

    plsc.subcore_barrier()

    def body(j, _):
        pltpu.async_copy(h_hbm.at[srcs_v.at[j]], rows_a, sem_a).wait()
        pltpu.sync_copy(rows_a, agg_sh.at[dsts_v.at[j]], add=True)
        return 0

    lax.fori_loop(0, NCHUNK, body, 0)

    plsc.subcore_barrier()
    pltpu.sync_copy(agg_sh.at[pl.ds(r0, RPT)], out_hbm.at[c, pl.ds(r0, RPT)])

    @pl.when(s == 0)
    def _out_tail():
        pltpu.sync_copy(agg_sh.at[pl.ds(NS * RPT, TAIL)],
                        out_hbm.at[c, pl.ds(NS * RPT, TAIL)])


BN = 1000


def _dense1_body(x_ref, a0_ref, a1_ref, wa_ref, ba_ref, wb_ref, bb_ref,
                 p0w_ref, p0b_ref, p1w_ref, p1b_ref, x1_ref, pred_ref):
    x = x_ref[...]
    hh = x + a0_ref[...] + a1_ref[...]
    t = jnp.maximum(jnp.dot(hh, wa_ref[...],
                            preferred_element_type=jnp.float32) + ba_ref[...],
                    0.0)
    x1 = jnp.maximum(jnp.dot(t, wb_ref[...],
                             preferred_element_type=jnp.float32) + bb_ref[...],
                     0.0)
    x1_ref[...] = x1
    pred_ref[...] = (
        jnp.dot(x, p0w_ref[...], preferred_element_type=jnp.float32)
        + p0b_ref[...]
        + jnp.dot(x1, p1w_ref[...], preferred_element_type=jnp.float32)
        + p1b_ref[...])


def _dense2_body(x_ref, a0_ref, a1_ref, wa_ref, ba_ref, wb_ref, bb_ref,
                 p2w_ref, p2b_ref, pin_ref, pred_ref):
    hh = x_ref[...] + a0_ref[...] + a1_ref[...]
    t = jnp.maximum(jnp.dot(hh, wa_ref[...],
                            preferred_element_type=jnp.float32) + ba_ref[...],
                    0.0)
    x2 = jnp.maximum(jnp.dot(t, wb_ref[...],
                             preferred_element_type=jnp.float32) + bb_ref[...],
                     0.0)
    pred_ref[...] = (
        pin_ref[...]
        + jnp.dot(x2, p2w_ref[...], preferred_element_type=jnp.float32)
        + p2b_ref[...])


def _row_spec(d):
    return pl.BlockSpec((BN, d), lambda i: (i, 0))


def _w_spec(a, b):
    return pl.BlockSpec((a, b), lambda i: (0, 0))


_dense1 = pl.pallas_call(
    _dense1_body,
    grid=(N // BN,),
    in_specs=[
        _row_spec(D), _row_spec(D), _row_spec(D),
        _w_spec(D, D), _w_spec(1, D), _w_spec(D, D), _w_spec(1, D),
        _w_spec(D, C), _w_spec(1, C), _w_spec(D, C), _w_spec(1, C),
    ],
    out_specs=[_row_spec(D), _row_spec(C)],
    out_shape=[
        jax.ShapeDtypeStruct((N, D), jnp.float32),
        jax.ShapeDtypeStruct((N, C), jnp.float32),
    ],
)

_dense2 = pl.pallas_call(
    _dense2_body,
    grid=(N // BN,),
    in_specs=[
        _row_spec(D), _row_spec(D), _row_spec(D),
        _w_spec(D, D), _w_spec(1, D), _w_spec(D, D), _w_spec(1, D),
        _w_spec(D, C), _w_spec(1, C), _row_spec(C),
    ],
    out_specs=_row_spec(C),
    out_shape=jax.ShapeDtypeStruct((N, C), jnp.float32),
)


@jax.jit
def kernel(X, edge_index, W1_0, b1_0, W1_1, b1_1, W2_0, b2_0, W2_1, b2_1,
           P0_W, P0_b, P1_W, P1_b, P2_W, P2_b):
    edge_src = edge_index[0].reshape(NW, NCHUNK, K)
    edge_dst = edge_index[1].reshape(NW, NCHUNK, K)
    zeros_nd = jnp.zeros((N, D), jnp.float32)

    r = lambda v: v.reshape(1, -1)

    agg1 = _sc_scatter(edge_src, edge_dst, X, zeros_nd)
    X1, pred01 = _dense1(X, agg1[0], agg1[1], W1_0, r(b1_0), W1_1, r(b1_1),
                         P0_W, r(P0_b), P1_W, r(P1_b))
    agg2 = _sc_scatter(edge_src, edge_dst, X1, zeros_nd)
    pred = _dense2(X1, agg2[0], agg2[1], W2_0, r(b2_0), W2_1, r(b2_1),
                   P2_W, r(P2_b), pred01)
    return pred
